# baseline (speedup 1.0000x reference)
import jax
import jax.numpy as jnp
from jax.experimental import pallas as pl
from jax.experimental.pallas import tpu as pltpu

_NS = 4
_K = 8192
_D = 32
_T = 16384
_TILE = 256
_NT = _T // _TILE


def _sumsq_rows(v):
    v = v * v
    acc = v[:, 0:8]
    for g in (1, 2, 3):
        acc = acc + v[:, g * 8:(g + 1) * 8]
    acc = acc[:, 0:4] + acc[:, 4:8]
    acc = acc[:, 0:2] + acc[:, 2:4]
    return acc[:, 0:1] + acc[:, 1:2]


def _pvq_kernel(x_ref, cb_ref, out_ref, mse_ref):
    i = pl.program_id(0)
    x0 = x_ref[...]
    x = x0
    recon = jnp.zeros_like(x0)
    iota = jax.lax.broadcasted_iota(jnp.int32, (_TILE, _K), 1)
    parts = []
    for s in range(_NS):
        cb = cb_ref[s]
        csum = _sumsq_rows(cb)[:, 0]
        rowsum = _sumsq_rows(x)
        xc = jax.lax.dot_general(
            x, cb, (((1,), (1,)), ((), ())),
            preferred_element_type=jnp.float32)
        dist = (rowsum - 2.0 * xc) + csum[None, :]
        if s == 0:
            half = _K // 2
            dlo, dhi = dist[:, :half], dist[:, half:]
            mlo = jnp.min(dlo, axis=1, keepdims=True)
            mhi = jnp.min(dhi, axis=1, keepdims=True)
            ilo = jnp.min(jnp.where(dlo == mlo, iota[:, :half], _K), axis=1)
            ihi = jnp.min(jnp.where(dhi == mhi, iota[:, half:], _K), axis=1)
            blo = jax.lax.bitcast_convert_type(mlo[:, 0], jnp.int32)
            bhi = jax.lax.bitcast_convert_type(mhi[:, 0], jnp.int32)
            tlo = jax.lax.shift_right_logical(blo, 16)
            thi = jax.lax.shift_right_logical(bhi, 16)
            b15 = jax.lax.shift_right_logical(blo, 15) & 1
            win_hi = (thi < tlo) | ((thi == tlo) & (b15 == 1))
            idx = jnp.where(win_hi, ihi, ilo)
        else:
            mv = jnp.min(dist, axis=1, keepdims=True)
            idx = jnp.min(jnp.where(dist == mv, iota, _K), axis=1)
        onehot = (iota == idx[:, None]).astype(jnp.float32)
        q = jax.lax.dot_general(
            onehot, cb, (((1,), (0,)), ((), ())),
            preferred_element_type=jnp.float32,
            precision=jax.lax.Precision.HIGHEST)
        x = x - q
        recon = recon + q
        parts.append(jnp.sum((recon - x0) ** 2))
    out_ref[...] = recon

    @pl.when(i == 0)
    def _():
        mse_ref[...] = jnp.zeros_like(mse_ref)

    mse_ref[...] += jnp.stack(parts)[None, :]

    @pl.when(i == _NT - 1)
    def _():
        mse_ref[...] = mse_ref[...] * (1.0 / (_T * _D))


def kernel(input_data, codebooks):
    final_output, mse = pl.pallas_call(
        _pvq_kernel,
        grid=(_NT,),
        in_specs=[
            pl.BlockSpec((_TILE, _D), lambda i: (i, 0)),
            pl.BlockSpec((_NS, _K, _D), lambda i: (0, 0, 0)),
        ],
        out_specs=[
            pl.BlockSpec((_TILE, _D), lambda i: (i, 0)),
            pl.BlockSpec((1, _NS), lambda i: (0, 0)),
        ],
        out_shape=[
            jax.ShapeDtypeStruct((_T, _D), jnp.float32),
            jax.ShapeDtypeStruct((1, _NS), jnp.float32),
        ],
        compiler_params=pltpu.CompilerParams(
            dimension_semantics=("arbitrary",),
        ),
    )(input_data, codebooks)
    codebooks_used = jnp.zeros((_NS, _K), dtype=jnp.int32)
    return (final_output, codebooks_used, codebooks, mse.reshape(_NS))


# speedup vs baseline: 1.0050x; 1.0050x over previous
import jax
import jax.numpy as jnp
from jax.experimental import pallas as pl
from jax.experimental.pallas import tpu as pltpu

_NS = 4
_K = 8192
_D = 32
_T = 16384
_TILE = 256
_NT = _T // _TILE


def _sumsq_rows(v):
    v = v * v
    acc = v[:, 0:8]
    for g in (1, 2, 3):
        acc = acc + v[:, g * 8:(g + 1) * 8]
    acc = acc[:, 0:4] + acc[:, 4:8]
    acc = acc[:, 0:2] + acc[:, 2:4]
    return acc[:, 0:1] + acc[:, 1:2]


def _pvq_kernel(x_ref, cb_ref, cbm2_ref, out_ref, mse_ref):
    x0 = x_ref[...]
    x = x0
    recon = jnp.zeros_like(x0)
    iota = jax.lax.broadcasted_iota(jnp.int32, (_TILE, _K), 1)
    parts = []
    for s in range(_NS):
        cb = cb_ref[s]
        csum = _sumsq_rows(cb)[:, 0]
        rowsum = _sumsq_rows(x)
        xc2 = jax.lax.dot_general(
            x, cbm2_ref[s], (((1,), (1,)), ((), ())),
            preferred_element_type=jnp.float32)
        dist = (rowsum + xc2) + csum[None, :]
        if s == 0:
            half = _K // 2
            dlo, dhi = dist[:, :half], dist[:, half:]
            mlo = jnp.min(dlo, axis=1, keepdims=True)
            mhi = jnp.min(dhi, axis=1, keepdims=True)
            ilo = jnp.min(jnp.where(dlo == mlo, iota[:, :half], _K), axis=1)
            ihi = jnp.min(jnp.where(dhi == mhi, iota[:, half:], _K), axis=1)
            blo = jax.lax.bitcast_convert_type(mlo[:, 0], jnp.int32)
            bhi = jax.lax.bitcast_convert_type(mhi[:, 0], jnp.int32)
            tlo = jax.lax.shift_right_logical(blo, 16)
            thi = jax.lax.shift_right_logical(bhi, 16)
            b15 = jax.lax.shift_right_logical(blo, 15) & 1
            win_hi = (thi < tlo) | ((thi == tlo) & (b15 == 1))
            idx = jnp.where(win_hi, ihi, ilo)
        else:
            mv = jnp.min(dist, axis=1, keepdims=True)
            idx = jnp.min(jnp.where(dist == mv, iota, _K), axis=1)
        onehot = (iota == idx[:, None]).astype(jnp.float32)
        q = jax.lax.dot_general(
            onehot, cb, (((1,), (0,)), ((), ())),
            preferred_element_type=jnp.float32,
            precision=jax.lax.Precision.HIGHEST)
        x = x - q
        recon = recon + q
        parts.append(jnp.sum((recon - x0) ** 2))
    out_ref[...] = recon
    mse_ref[...] = jnp.stack(parts)[None, None, :]


def kernel(input_data, codebooks):
    cbm2 = -2.0 * codebooks
    final_output, mse_parts = pl.pallas_call(
        _pvq_kernel,
        grid=(_NT,),
        in_specs=[
            pl.BlockSpec((_TILE, _D), lambda i: (i, 0)),
            pl.BlockSpec((_NS, _K, _D), lambda i: (0, 0, 0)),
            pl.BlockSpec((_NS, _K, _D), lambda i: (0, 0, 0)),
        ],
        out_specs=[
            pl.BlockSpec((_TILE, _D), lambda i: (i, 0)),
            pl.BlockSpec((1, 1, _NS), lambda i: (i, 0, 0)),
        ],
        out_shape=[
            jax.ShapeDtypeStruct((_T, _D), jnp.float32),
            jax.ShapeDtypeStruct((_NT, 1, _NS), jnp.float32),
        ],
        compiler_params=pltpu.CompilerParams(
            dimension_semantics=("parallel",),
        ),
    )(input_data, codebooks, cbm2)
    mse = mse_parts.reshape(_NT, _NS).sum(axis=0) * (1.0 / (_T * _D))
    codebooks_used = jnp.zeros((_NS, _K), dtype=jnp.int32)
    return (final_output, codebooks_used, codebooks, mse)


# speedup vs baseline: 1.1293x; 1.1237x over previous
import jax
import jax.numpy as jnp
from jax.experimental import pallas as pl
from jax.experimental.pallas import tpu as pltpu

_NS = 4
_K = 8192
_D = 32
_T = 16384
_TILE = 512
_NT = _T // _TILE


def _sumsq_rows(v):
    v = v * v
    acc = v[:, 0:8]
    for g in (1, 2, 3):
        acc = acc + v[:, g * 8:(g + 1) * 8]
    acc = acc[:, 0:4] + acc[:, 4:8]
    acc = acc[:, 0:2] + acc[:, 2:4]
    return acc[:, 0:1] + acc[:, 1:2]


def _pvq_kernel(x_ref, cb_ref, cbm2_ref, out_ref, mse_ref):
    x0 = x_ref[...]
    x = x0
    recon = jnp.zeros_like(x0)
    iota = jax.lax.broadcasted_iota(jnp.int32, (_TILE, _K), 1)
    parts = []
    for s in range(_NS):
        cb = cb_ref[s]
        csum = _sumsq_rows(cb)[:, 0]
        rowsum = _sumsq_rows(x)
        xc2 = jax.lax.dot_general(
            x, cbm2_ref[s], (((1,), (1,)), ((), ())),
            preferred_element_type=jnp.float32)
        dist = (rowsum + xc2) + csum[None, :]
        if s == 0:
            half = _K // 2
            dlo, dhi = dist[:, :half], dist[:, half:]
            mlo = jnp.min(dlo, axis=1, keepdims=True)
            mhi = jnp.min(dhi, axis=1, keepdims=True)
            ilo = jnp.min(jnp.where(dlo == mlo, iota[:, :half], _K), axis=1)
            ihi = jnp.min(jnp.where(dhi == mhi, iota[:, half:], _K), axis=1)
            blo = jax.lax.bitcast_convert_type(mlo[:, 0], jnp.int32)
            bhi = jax.lax.bitcast_convert_type(mhi[:, 0], jnp.int32)
            tlo = jax.lax.shift_right_logical(blo, 16)
            thi = jax.lax.shift_right_logical(bhi, 16)
            b15 = jax.lax.shift_right_logical(blo, 15) & 1
            win_hi = (thi < tlo) | ((thi == tlo) & (b15 == 1))
            idx = jnp.where(win_hi, ihi, ilo)
        else:
            mv = jnp.min(dist, axis=1, keepdims=True)
            idx = jnp.min(jnp.where(dist == mv, iota, _K), axis=1)
        onehot = (iota == idx[:, None]).astype(jnp.float32)
        q = jax.lax.dot_general(
            onehot, cb, (((1,), (0,)), ((), ())),
            preferred_element_type=jnp.float32,
            precision=jax.lax.Precision.HIGHEST)
        x = x - q
        recon = recon + q
        parts.append(jnp.sum((recon - x0) ** 2))
    out_ref[...] = recon
    mse_ref[...] = jnp.stack(parts)[None, None, :]


def kernel(input_data, codebooks):
    cbm2 = -2.0 * codebooks
    final_output, mse_parts = pl.pallas_call(
        _pvq_kernel,
        grid=(_NT,),
        in_specs=[
            pl.BlockSpec((_TILE, _D), lambda i: (i, 0)),
            pl.BlockSpec((_NS, _K, _D), lambda i: (0, 0, 0)),
            pl.BlockSpec((_NS, _K, _D), lambda i: (0, 0, 0)),
        ],
        out_specs=[
            pl.BlockSpec((_TILE, _D), lambda i: (i, 0)),
            pl.BlockSpec((1, 1, _NS), lambda i: (i, 0, 0)),
        ],
        out_shape=[
            jax.ShapeDtypeStruct((_T, _D), jnp.float32),
            jax.ShapeDtypeStruct((_NT, 1, _NS), jnp.float32),
        ],
        compiler_params=pltpu.CompilerParams(
            dimension_semantics=("parallel",),
        ),
    )(input_data, codebooks, cbm2)
    mse = mse_parts.reshape(_NT, _NS).sum(axis=0) * (1.0 / (_T * _D))
    codebooks_used = jnp.zeros((_NS, _K), dtype=jnp.int32)
    return (final_output, codebooks_used, codebooks, mse)
